# baseline (speedup 1.0000x reference)
import functools

import jax
import jax.numpy as jnp
from jax import lax
from jax.experimental import pallas as pl
from jax.experimental.pallas import tpu as pltpu
from jax.experimental.pallas import tpu_sc as plsc

EMBED_DIM = 32


@functools.cache
def _make_gather(n_outer: int, n_inner: int, vocab: int):
    info = plsc.get_sparse_core_info()
    nc, ns = info.num_cores, info.num_subcores
    nw = nc * ns
    outer_per_w = n_outer // nw
    rows_per_w = outer_per_w * n_inner
    chunk_o = 32
    chunk = chunk_o * n_inner
    n_chunks = outer_per_w // chunk_o
    nbuf = 3
    assert outer_per_w % chunk_o == 0 and n_outer % nw == 0

    mesh = plsc.VectorSubcoreMesh(core_axis_name="c", subcore_axis_name="s")

    @functools.partial(
        pl.kernel,
        mesh=mesh,
        out_type=jax.ShapeDtypeStruct((n_outer, n_inner, EMBED_DIM), jnp.float32),
        scratch_types=[
            pltpu.VMEM((rows_per_w,), jnp.int32),
            pltpu.VMEM((nbuf, chunk, EMBED_DIM), jnp.float32),
            [pltpu.SemaphoreType.DMA] * nbuf,
            [pltpu.SemaphoreType.DMA] * nbuf,
        ],
        compiler_params=pltpu.CompilerParams(use_tc_tiling_on_sc=False),
    )
    def gather_kernel(idx_hbm, table_hbm, out_hbm, idx_v, rows_v, gsems, ssems):
        wid = lax.axis_index("s") * nc + lax.axis_index("c")
        base = wid * rows_per_w
        obase = wid * outer_per_w
        pltpu.sync_copy(idx_hbm.at[pl.ds(base, rows_per_w)], idx_v)

        gathers = [None] * n_chunks

        def start_gather(c):
            b = c % nbuf
            gathers[c] = pltpu.async_copy(
                table_hbm.at[idx_v.at[pl.ds(c * chunk, chunk)]],
                rows_v.at[b],
                gsems[b],
            )

        def start_store(c):
            b = c % nbuf
            gathers[c].wait()

            def body(j, carry):
                pltpu.async_copy(
                    rows_v.at[b, pl.ds(j * n_inner, n_inner)],
                    out_hbm.at[obase + c * chunk_o + j],
                    ssems[b],
                )
                return carry

            lax.fori_loop(0, chunk_o, body, 0)

        def wait_store(c):
            b = c % nbuf

            def body(j, carry):
                pltpu.make_async_copy(
                    rows_v.at[b, pl.ds(j * n_inner, n_inner)],
                    out_hbm.at[obase + c * chunk_o + j],
                    ssems[b],
                ).wait()
                return carry

            lax.fori_loop(0, chunk_o, body, 0)

        for c in range(n_chunks):
            if c >= nbuf:
                wait_store(c - nbuf)
            start_gather(c)
            if c >= 1:
                start_store(c - 1)
        start_store(n_chunks - 1)
        for c in range(n_chunks - nbuf, n_chunks):
            wait_store(c)

    return gather_kernel


def kernel(inputs, embed_table):
    b, s = inputs.shape
    idx = inputs.reshape(-1).astype(jnp.int32)
    return _make_gather(b, s, embed_table.shape[0])(idx, embed_table)
